# baseline (speedup 1.0000x reference)
import jax
import jax.numpy as jnp
from jax.experimental import pallas as pl
from jax.experimental.pallas import tpu as pltpu

BM = 1024
NBUF = 3


def _mp_kern(x_hbm, w_hbm, m_hbm, b_hbm, o_hbm,
             wvm, mvm, bvm, wm, xbuf, obuf,
             wsems, in_sems, out_sems):
    M = x_hbm.shape[0]
    T = M // BM

    w_cp = pltpu.make_async_copy(w_hbm, wvm, wsems.at[0])
    m_cp = pltpu.make_async_copy(m_hbm, mvm, wsems.at[1])
    b_cp = pltpu.make_async_copy(b_hbm, bvm, wsems.at[2])
    w_cp.start()
    m_cp.start()
    b_cp.start()
    HB2 = BM // 2

    def start_x(t, buf):
        if t == 0 or t == T - 1:
            cps = []
            for q in range(2):
                cp = pltpu.make_async_copy(
                    x_hbm.at[pl.ds(t * BM + q * HB2, HB2), :],
                    xbuf.at[buf, pl.ds(q * HB2, HB2), :],
                    in_sems.at[buf, q])
                cp.start()
                cps.append(cp)
            return cps
        cp = pltpu.make_async_copy(
            x_hbm.at[pl.ds(t * BM, BM), :],
            xbuf.at[buf, pl.ds(0, BM), :],
            in_sems.at[buf, 0])
        cp.start()
        return [cp]

    x_cps = []
    for t in range(min(NBUF, T)):
        x_cps.append(start_x(t, t % NBUF))

    w_cp.wait()
    m_cp.wait()
    wm[...] = (wvm[...] * mvm[...]).astype(jnp.bfloat16)
    b_cp.wait()

    out_cps = [[None] * 4 for _ in range(NBUF)]
    for t in range(T):
        buf = t % NBUF
        nh = 4 if t == T - 1 else 2
        hb = BM // nh
        split = (t == 0 or t == T - 1)
        if not split:
            x_cps[t][0].wait()
        for h in range(4):
            if out_cps[buf][h] is not None:
                out_cps[buf][h].wait()
                out_cps[buf][h] = None
        for h in range(nh):
            if split and (h == 0 or h * hb == HB2):
                x_cps[t][h * hb // HB2].wait()
            xbh = xbuf[buf, h * hb:(h + 1) * hb, :].astype(jnp.bfloat16)
            acc = jax.lax.dot_general(
                xbh, wm[...],
                dimension_numbers=(((1,), (1,)), ((), ())),
                preferred_element_type=jnp.float32,
            )
            obuf[buf, h * hb:(h + 1) * hb, :] = acc + bvm[...]
            ocp = pltpu.make_async_copy(
                obuf.at[buf, h * hb:(h + 1) * hb, :],
                o_hbm.at[(t * BM + h * hb):(t * BM + (h + 1) * hb), :],
                out_sems.at[buf, h])
            ocp.start()
            out_cps[buf][h] = ocp
        nxt = t + NBUF
        if nxt < T:
            x_cps.append(start_x(nxt, buf))

    for buf in range(min(NBUF, T)):
        for h in range(4):
            if out_cps[buf][h] is not None:
                out_cps[buf][h].wait()


def _masked_linear(x2, W, b2, binary_mask):
    M, K = x2.shape
    N = W.shape[0]
    return pl.pallas_call(
        _mp_kern,
        in_specs=[
            pl.BlockSpec(memory_space=pl.ANY),
            pl.BlockSpec(memory_space=pl.ANY),
            pl.BlockSpec(memory_space=pl.ANY),
            pl.BlockSpec(memory_space=pl.ANY),
        ],
        out_specs=pl.BlockSpec(memory_space=pl.ANY),
        out_shape=jax.ShapeDtypeStruct((M, N), jnp.float32),
        scratch_shapes=[
            pltpu.VMEM((N, K), jnp.float32),
            pltpu.VMEM((N, K), jnp.float32),
            pltpu.VMEM((1, N), jnp.float32),
            pltpu.VMEM((N, K), jnp.bfloat16),
            pltpu.VMEM((NBUF, BM, K), jnp.float32),
            pltpu.VMEM((NBUF, BM, N), jnp.float32),
            pltpu.SemaphoreType.DMA((3,)),
            pltpu.SemaphoreType.DMA((NBUF, 2)),
            pltpu.SemaphoreType.DMA((NBUF, 4)),
        ],
    )(x2, W, binary_mask, b2)


def kernel(x, W, b, binary_mask):
    B, S, D = x.shape
    N = W.shape[0]
    out = _masked_linear(x.reshape(B * S, D), W, b.reshape(1, N), binary_mask)
    return out.reshape(B, S, N)


# speedup vs baseline: 1.0292x; 1.0292x over previous
import jax
import jax.numpy as jnp
from jax.experimental import pallas as pl
from jax.experimental.pallas import tpu as pltpu

BM = 1024
NBUF = 3


def _mp_kern(x_hbm, w_hbm, m_hbm, b_hbm, o_hbm,
             wvm, mvm, bvm, wm, xbuf, obuf,
             wsems, in_sems, out_sems):
    M = x_hbm.shape[0]
    T = M // BM

    w_cp = pltpu.make_async_copy(w_hbm, wvm, wsems.at[0])
    m_cp = pltpu.make_async_copy(m_hbm, mvm, wsems.at[1])
    b_cp = pltpu.make_async_copy(b_hbm, bvm, wsems.at[2])
    w_cp.start()
    m_cp.start()
    b_cp.start()
    x_cps = []
    for t in range(min(NBUF, T)):
        cp = pltpu.make_async_copy(
            x_hbm.at[pl.ds(t * BM, BM), :], xbuf.at[t % NBUF], in_sems.at[t % NBUF])
        cp.start()
        x_cps.append(cp)

    w_cp.wait()
    m_cp.wait()
    wm[...] = (wvm[...] * mvm[...]).astype(jnp.bfloat16)
    b_cp.wait()

    out_cps = [[None] * 4 for _ in range(NBUF)]
    for t in range(T):
        buf = t % NBUF
        nh = 4 if t == T - 1 else 2
        hb = BM // nh
        x_cps[t].wait()
        for h in range(4):
            if out_cps[buf][h] is not None:
                out_cps[buf][h].wait()
                out_cps[buf][h] = None
        xb = xbuf[buf].astype(jnp.bfloat16)
        for h in range(nh):
            acc = jax.lax.dot_general(
                xb[h * hb:(h + 1) * hb, :], wm[...],
                dimension_numbers=(((1,), (1,)), ((), ())),
                preferred_element_type=jnp.float32,
            )
            obuf[buf, h * hb:(h + 1) * hb, :] = acc + bvm[...]
            ocp = pltpu.make_async_copy(
                obuf.at[buf, h * hb:(h + 1) * hb, :],
                o_hbm.at[(t * BM + h * hb):(t * BM + (h + 1) * hb), :],
                out_sems.at[buf, h])
            ocp.start()
            out_cps[buf][h] = ocp
        nxt = t + NBUF
        if nxt < T:
            cp = pltpu.make_async_copy(
                x_hbm.at[pl.ds(nxt * BM, BM), :], xbuf.at[buf], in_sems.at[buf])
            cp.start()
            x_cps.append(cp)

    for buf in range(min(NBUF, T)):
        for h in range(4):
            if out_cps[buf][h] is not None:
                out_cps[buf][h].wait()


def _masked_linear(x2, W, b2, binary_mask):
    M, K = x2.shape
    N = W.shape[0]
    return pl.pallas_call(
        _mp_kern,
        in_specs=[
            pl.BlockSpec(memory_space=pl.ANY),
            pl.BlockSpec(memory_space=pl.ANY),
            pl.BlockSpec(memory_space=pl.ANY),
            pl.BlockSpec(memory_space=pl.ANY),
        ],
        out_specs=pl.BlockSpec(memory_space=pl.ANY),
        out_shape=jax.ShapeDtypeStruct((M, N), jnp.float32),
        scratch_shapes=[
            pltpu.VMEM((N, K), jnp.float32),
            pltpu.VMEM((N, K), jnp.float32),
            pltpu.VMEM((1, N), jnp.float32),
            pltpu.VMEM((N, K), jnp.bfloat16),
            pltpu.VMEM((NBUF, BM, K), jnp.float32),
            pltpu.VMEM((NBUF, BM, N), jnp.float32),
            pltpu.SemaphoreType.DMA((3,)),
            pltpu.SemaphoreType.DMA((NBUF,)),
            pltpu.SemaphoreType.DMA((NBUF, 4)),
        ],
    )(x2, W, binary_mask, b2)


def kernel(x, W, b, binary_mask):
    B, S, D = x.shape
    N = W.shape[0]
    out = _masked_linear(x.reshape(B * S, D), W, b.reshape(1, N), binary_mask)
    return out.reshape(B, S, N)
